# baseline (speedup 1.0000x reference)
import functools
import math

import jax
import jax.numpy as jnp
from jax import lax
from jax.experimental import pallas as pl
from jax.experimental.pallas import tpu as pltpu
from jax.experimental.pallas import tpu_sc as plsc

VOCAB = 1_000_000
EMB = 64
B = 16384
L = 50
SCALE = math.sqrt(EMB)

NC = 2
NS = 16
LANES = 16

E_PER_CORE = EMB // NC
B_PER_SUB = B // NS
CHUNK = 128
N_CHUNK = B_PER_SUB // CHUNK
N_STAGERS = 8
_TAIL = 128
_MAIN_TILES = (VOCAB - VOCAB % 128) // 128
_tiles = [_MAIN_TILES // N_STAGERS + (1 if i < _MAIN_TILES % N_STAGERS else 0)
          for i in range(N_STAGERS)]
_STAGE_SIZES = [t * 128 for t in _tiles]
_STAGE_OFFS = [sum(_STAGE_SIZES[:i]) for i in range(N_STAGERS)]


def _make_kernel():
    mesh = plsc.VectorSubcoreMesh(core_axis_name="c", subcore_axis_name="s")

    @functools.partial(
        pl.kernel,
        out_type=jax.ShapeDtypeStruct((L, EMB, B), jnp.float32),
        mesh=mesh,
        scratch_types=[
            pltpu.VMEM((L, B_PER_SUB), jnp.int32),
            pltpu.VMEM((B_PER_SUB,), jnp.float32),
            pltpu.VMEM((B_PER_SUB,), jnp.float32),
            pltpu.VMEM((128,), jnp.float32),
            pltpu.VMEM_SHARED((VOCAB,), jnp.float32),
            pltpu.SemaphoreType.DMA,
            pltpu.SemaphoreType.DMA,
            pltpu.SemaphoreType.DMA,
            pltpu.SemaphoreType.DMA,
            pltpu.SemaphoreType.DMA,
            pltpu.SemaphoreType.DMA,
        ],
        compiler_params=pltpu.CompilerParams(use_tc_tiling_on_sc=True),
    )
    def k(tok_hbm, tab_hbm, tail_hbm, out_hbm, tok_v, gbuf0, gbuf1, tbuf,
          row, sem_t, sem_r, sem_g0, sem_g1, sem_o0, sem_o1):
        cid = lax.axis_index("c")
        sid = lax.axis_index("s")
        e_base = cid * E_PER_CORE
        b_base = sid * B_PER_SUB

        gbufs = (gbuf0, gbuf1)
        sem_gs = (sem_g0, sem_g1)
        sem_os = (sem_o0, sem_o1)

        pltpu.make_async_copy(tok_hbm.at[sid], tok_v, sem_t).start()

        def row_desc(k_e, i):
            off, sz = _STAGE_OFFS[i], _STAGE_SIZES[i]
            return pltpu.make_async_copy(
                tab_hbm.at[e_base + k_e].at[pl.ds(off, sz)],
                row.at[pl.ds(off, sz)],
                sem_r)

        def tail_desc(k_e):
            return pltpu.make_async_copy(
                tail_hbm.at[pl.ds((e_base + k_e) * _TAIL, _TAIL)],
                tbuf, sem_r)

        def _tail_wait(k_e):
            tail_desc(k_e).wait()
            pltpu.sync_copy(tbuf, row.at[pl.ds(VOCAB - _TAIL, _TAIL)])

        def row_stage(k_e, op):
            for i in range(N_STAGERS):
                pl.when(sid == i)(lambda i=i: getattr(row_desc(k_e, i), op)())
            if op == "start":
                pl.when(sid == N_STAGERS)(lambda: tail_desc(k_e).start())
            else:
                pl.when(sid == N_STAGERS)(lambda: _tail_wait(k_e))

        def gather_descs(row_buf, l, gbuf, sem):
            return [
                pltpu.make_async_copy(
                    row_buf.at[tok_v.at[l, pl.ds(j * CHUNK, CHUNK)]],
                    gbuf.at[pl.ds(j * CHUNK, CHUNK)],
                    sem)
                for j in range(N_CHUNK)
            ]

        def out_desc(l, k_e, gbuf, sem):
            return pltpu.make_async_copy(
                gbuf, out_hbm.at[l, e_base + k_e, pl.ds(b_base, B_PER_SUB)],
                sem)

        row_stage(0, "start")

        pltpu.make_async_copy(tok_hbm.at[sid], tok_v, sem_t).wait()

        @pl.loop(0, E_PER_CORE)
        def _(k_e):
            row_stage(k_e, "wait")

            plsc.subcore_barrier()

            for d in gather_descs(row, 0, gbufs[0], sem_gs[0]):
                d.start()

            @pl.loop(0, L, step=2)
            def _(l0):
                for b2 in range(2):
                    l = l0 + b2
                    gbuf, sem_g = gbufs[b2], sem_gs[b2]
                    obuf, osem_g = gbufs[1 - b2], sem_gs[1 - b2]

                    @pl.when(l + 1 < L)
                    def _():
                        @pl.when(l >= 1)
                        def _():
                            out_desc(l - 1, k_e, obuf, sem_os[1 - b2]).wait()

                        for d in gather_descs(row, l + 1, obuf, osem_g):
                            d.start()

                    for d in gather_descs(row, l, gbuf, sem_g):
                        d.wait()

                    @plsc.parallel_loop(0, B_PER_SUB, step=LANES, unroll=4)
                    def _(i):
                        sl = pl.ds(i, LANES)
                        gbuf[sl] = gbuf[sl] * SCALE

                    out_desc(l, k_e, gbuf, sem_os[b2]).start()

            out_desc(L - 2, k_e, gbufs[0], sem_os[0]).wait()
            out_desc(L - 1, k_e, gbufs[1], sem_os[1]).wait()

            plsc.subcore_barrier()

            @pl.when(k_e + 1 < E_PER_CORE)
            def _():
                row_stage(k_e + 1, "start")

    return k


_K = _make_kernel()


@jax.jit
def kernel(tokens, table):
    tok_t = jnp.transpose(tokens)
    tok_blk = jnp.transpose(jnp.reshape(tok_t, (L, NS, B_PER_SUB)), (1, 0, 2))
    tab_t = jnp.transpose(table)
    tab_tail = jnp.reshape(
        lax.slice(tab_t, (0, VOCAB - _TAIL), (EMB, VOCAB)), (EMB * _TAIL,))
    out_k = _K(tok_blk, tab_t, tab_tail)
    return jnp.transpose(out_k, (2, 0, 1))
